# baseline (speedup 1.0000x reference)
import functools

import jax
import jax.numpy as jnp
from jax import lax
from jax.experimental import pallas as pl
from jax.experimental.pallas import tpu as pltpu
from jax.experimental.pallas import tpu_sc as plsc

B = 3
D = 64
H = 32
W = 128
CW = 2
CH = W * CW
K = 6
NJ = 126
NTILES = 16
EPAD = NTILES * NJ * CH
SL = 3120
ZR = 80


def _proj_body(x_ref, w_ref, o_ref):
    res = jnp.dot(x_ref[...], w_ref[...], preferred_element_type=jnp.float32)
    o_ref[0] = res[:, :H]
    o_ref[1] = res[:, H:]


def _project(x, w, block=2000):
    v = x.shape[0]
    return pl.pallas_call(
        _proj_body,
        grid=(v // block,),
        in_specs=[
            pl.BlockSpec((block, D), lambda i: (i, 0)),
            pl.BlockSpec((D, D), lambda i: (0, 0)),
        ],
        out_specs=pl.BlockSpec((2, block, H), lambda i: (0, i, 0)),
        out_shape=jax.ShapeDtypeStruct((2, v, H), jnp.float32),
    )(x, w)


def _sc_body(nv, p_item, p_user, pack, agg_u, agg_i,
             acc, rows, pk, zeros, sem):
    c = lax.axis_index("c")
    s = lax.axis_index("s")
    rem = nv - NTILES * SL

    def zfill(r, zc):
        z = jnp.zeros((16,), jnp.float32)
        zeros[r, 0:16] = z
        zeros[r, 16:32] = z
        return zc
    lax.fori_loop(0, ZR, zfill, 0)

    def one_pass(b, table, splane, dplane, out):
        for k in range(SL // ZR):
            pltpu.sync_copy(zeros, acc.at[pl.ds(s * SL + k * ZR, ZR)])
        if rem:
            @pl.when(s == 0)
            def _():
                pltpu.sync_copy(zeros.at[pl.ds(0, rem)],
                                acc.at[pl.ds(NTILES * SL, rem)])
        plsc.subcore_barrier()

        base = s * (NJ * CW)
        pltpu.sync_copy(pack.at[b, pl.ds(base, K * CW)], pk.at[0])
        for w in range(CW):
            pltpu.async_copy(table.at[c].at[pk.at[0, w, splane]],
                             rows.at[0, pl.ds(w * W, W)], sem)

        def chunk_body(j, carry):
            q = j % 2
            within = j % K
            pq = (j // K) % 2

            @pl.when((within == K - 1) & (j < NJ - K))
            def _():
                sup1 = j // K + 1
                pltpu.sync_copy(
                    pack.at[b, pl.ds(base + sup1 * (K * CW), K * CW)],
                    pk.at[sup1 % 2])

            @pl.when(j < NJ - 1)
            def _():
                jn = j + 1
                pqn = (jn // K) % 2
                wn = jn % K
                for w in range(CW):
                    pltpu.async_copy(
                        table.at[c].at[pk.at[pqn, wn * CW + w, splane]],
                        rows.at[1 - q, pl.ds(w * W, W)], sem)

            for w in range(CW):
                pltpu.make_async_copy(
                    table.at[c].at[pk.at[pq, within * CW + w, splane]],
                    rows.at[q, pl.ds(w * W, W)], sem).wait()

            def scale_body(g, sc_):
                wq = within * CW + g // (W // 16)
                off = (g % (W // 16)) * 16
                vg = plsc.bitcast(pk[pq, wq, 2, pl.ds(off, 16)], jnp.float32)
                base_r = g * 16
                for e in range(16):
                    bc = lax.gather(
                        vg, jnp.full((16, 1), e, jnp.int32),
                        _GATHER_DNUMS, (1,),
                        mode=lax.GatherScatterMode.PROMISE_IN_BOUNDS)
                    r = base_r + e
                    rows[q, r, 0:16] = rows[q, r, 0:16] * bc
                    rows[q, r, 16:32] = rows[q, r, 16:32] * bc
                return sc_
            lax.fori_loop(0, CH // 16, scale_body, 0)
            for w in range(CW):
                pltpu.sync_copy(rows.at[q, pl.ds(w * W, W)],
                                acc.at[pk.at[pq, within * CW + w, dplane]],
                                add=True)
            return carry
        lax.fori_loop(0, NJ, chunk_body, 0)
        plsc.subcore_barrier()
        pltpu.sync_copy(acc.at[pl.ds(s * SL, SL)],
                        out.at[b, c, pl.ds(s * SL, SL)])
        if rem:
            @pl.when(s == 0)
            def _():
                pltpu.sync_copy(acc.at[pl.ds(NTILES * SL, rem)],
                                out.at[b, c, pl.ds(NTILES * SL, rem)])

    for b in range(B):
        one_pass(b, p_item, 1, 0, agg_u)
        one_pass(b, p_user, 0, 1, agg_i)


_GATHER_DNUMS = lax.GatherDimensionNumbers(
    offset_dims=(), collapsed_slice_dims=(0,), start_index_map=(0,))


def _sc_aggregate(p_item, p_user, e_user, e_item, e_val):
    nu = p_user.shape[1]
    ni = p_item.shape[1]
    ne0 = e_user.shape[1]
    pad = EPAD - ne0
    if pad:
        e_user = jnp.pad(e_user, ((0, 0), (0, pad)))
        e_item = jnp.pad(e_item, ((0, 0), (0, pad)))
        e_val = jnp.pad(e_val, ((0, 0), (0, pad)))
    pack = jnp.stack(
        [e_user.reshape(B, EPAD // W, W),
         e_item.reshape(B, EPAD // W, W),
         lax.bitcast_convert_type(e_val, jnp.int32).reshape(B, EPAD // W, W)],
        axis=2)
    mesh = plsc.VectorSubcoreMesh(core_axis_name="c", subcore_axis_name="s")
    f = pl.kernel(
        functools.partial(_sc_body, nu),
        out_type=(
            jax.ShapeDtypeStruct((B, 2, nu, H), jnp.float32),
            jax.ShapeDtypeStruct((B, 2, ni, H), jnp.float32),
        ),
        mesh=mesh,
        scratch_types=[
            pltpu.VMEM_SHARED((nu, H), jnp.float32),
            pltpu.VMEM((2, CH, H), jnp.float32),
            pltpu.VMEM((2, K * CW, 3, W), jnp.int32),
            pltpu.VMEM((ZR, H), jnp.float32),
            pltpu.SemaphoreType.DMA,
        ],
        compiler_params=pltpu.CompilerParams(use_tc_tiling_on_sc=False,
                                             needs_layout_passes=False),
    )
    return f(p_item, p_user, pack)


def _post_body(a0_ref, a1_ref, embs_ref, emb_ref):
    a = jnp.concatenate([a0_ref[:, 0], a1_ref[:, 0]], axis=-1)
    embs_ref[...] = jax.nn.sigmoid(a)
    emb_ref[...] = jax.nn.sigmoid(jnp.mean(a, axis=0))


def _post(agg, block=2000):
    v = agg.shape[2]
    return pl.pallas_call(
        _post_body,
        grid=(v // block,),
        in_specs=[
            pl.BlockSpec((B, 1, block, H), lambda i: (0, 0, i, 0)),
            pl.BlockSpec((B, 1, block, H), lambda i: (0, 1, i, 0)),
        ],
        out_specs=[
            pl.BlockSpec((B, block, D), lambda i: (0, i, 0)),
            pl.BlockSpec((block, D), lambda i: (i, 0)),
        ],
        out_shape=[
            jax.ShapeDtypeStruct((B, v, D), jnp.float32),
            jax.ShapeDtypeStruct((v, D), jnp.float32),
        ],
    )(agg, agg)


def kernel(user_embedding, item_embedding, u_w, i_w, edge_user, edge_item,
           edge_val):
    p_item = _project(item_embedding, u_w)
    p_user = _project(user_embedding, i_w)
    agg_u, agg_i = _sc_aggregate(p_item, p_user, edge_user, edge_item,
                                 edge_val)
    user_embs, user_emb = _post(agg_u)
    item_embs, item_emb = _post(agg_i)
    return (user_emb, item_emb, user_embs, item_embs)


# speedup vs baseline: 1.2457x; 1.2457x over previous
import functools

import jax
import jax.numpy as jnp
from jax import lax
from jax.experimental import pallas as pl
from jax.experimental.pallas import tpu as pltpu
from jax.experimental.pallas import tpu_sc as plsc

B = 3
D = 64
H = 32
W = 128
CW = 6
CH = W * CW
NJ = 41
NTILES = 16
EPAD = NTILES * NJ * CH
SL = 3120
ZR = 80


def _proj_body(x_ref, w_ref, o_ref):
    res = jnp.dot(x_ref[...], w_ref[...], preferred_element_type=jnp.float32)
    o_ref[0] = res[:, :H]
    o_ref[1] = res[:, H:]


def _project(x, w, block=2000):
    v = x.shape[0]
    return pl.pallas_call(
        _proj_body,
        grid=(v // block,),
        in_specs=[
            pl.BlockSpec((block, D), lambda i: (i, 0)),
            pl.BlockSpec((D, D), lambda i: (0, 0)),
        ],
        out_specs=pl.BlockSpec((2, block, H), lambda i: (0, i, 0)),
        out_shape=jax.ShapeDtypeStruct((2, v, H), jnp.float32),
    )(x, w)


def _sc_body(nv, p_item, p_user, pack, agg_u, agg_i,
             acc, rows, pk, zeros, sem):
    c = lax.axis_index("c")
    s = lax.axis_index("s")
    rem = nv - NTILES * SL

    def zfill(r, zc):
        z = jnp.zeros((16,), jnp.float32)
        zeros[r, 0:16] = z
        zeros[r, 16:32] = z
        return zc
    lax.fori_loop(0, ZR, zfill, 0)

    def one_pass(b, table, splane, dplane, out):
        for k in range(SL // ZR):
            pltpu.sync_copy(zeros, acc.at[pl.ds(s * SL + k * ZR, ZR)])
        if rem:
            @pl.when(s == 0)
            def _():
                pltpu.sync_copy(zeros.at[pl.ds(0, rem)],
                                acc.at[pl.ds(NTILES * SL, rem)])
        plsc.subcore_barrier()

        def chunk_body(j, carry):
            chunk = s + NTILES * j
            pltpu.sync_copy(pack.at[b, chunk], pk)
            pltpu.async_copy(table.at[c].at[pk.at[splane]],
                             rows, sem).wait()

            def scale_body(g, sc_):
                vg = plsc.bitcast(pk[2, pl.ds(g * 16, 16)], jnp.float32)
                base = g * 16
                for e in range(16):
                    bc = lax.gather(
                        vg, jnp.full((16, 1), e, jnp.int32),
                        _GATHER_DNUMS, (1,),
                        mode=lax.GatherScatterMode.PROMISE_IN_BOUNDS)
                    r = base + e
                    rows[r, 0:16] = rows[r, 0:16] * bc
                    rows[r, 16:32] = rows[r, 16:32] * bc
                return sc_
            lax.fori_loop(0, CH // 16, scale_body, 0)
            pltpu.sync_copy(rows, acc.at[pk.at[dplane]], add=True)
            return carry
        lax.fori_loop(0, NJ, chunk_body, 0)
        plsc.subcore_barrier()
        pltpu.sync_copy(acc.at[pl.ds(s * SL, SL)],
                        out.at[b, c, pl.ds(s * SL, SL)])
        if rem:
            @pl.when(s == 0)
            def _():
                pltpu.sync_copy(acc.at[pl.ds(NTILES * SL, rem)],
                                out.at[b, c, pl.ds(NTILES * SL, rem)])

    for b in range(B):
        one_pass(b, p_item, 1, 0, agg_u)
        one_pass(b, p_user, 0, 1, agg_i)


_GATHER_DNUMS = lax.GatherDimensionNumbers(
    offset_dims=(), collapsed_slice_dims=(0,), start_index_map=(0,))


def _sc_aggregate(p_item, p_user, e_user, e_item, e_val):
    nu = p_user.shape[1]
    ni = p_item.shape[1]
    ne0 = e_user.shape[1]
    pad = EPAD - ne0
    if pad:
        e_user = jnp.pad(e_user, ((0, 0), (0, pad)))
        e_item = jnp.pad(e_item, ((0, 0), (0, pad)))
        e_val = jnp.pad(e_val, ((0, 0), (0, pad)))
    nc = EPAD // CH
    pack = jnp.stack(
        [e_user.reshape(B, nc, CH),
         e_item.reshape(B, nc, CH),
         lax.bitcast_convert_type(e_val, jnp.int32).reshape(B, nc, CH)],
        axis=2)
    mesh = plsc.VectorSubcoreMesh(core_axis_name="c", subcore_axis_name="s")
    f = pl.kernel(
        functools.partial(_sc_body, nu),
        out_type=(
            jax.ShapeDtypeStruct((B, 2, nu, H), jnp.float32),
            jax.ShapeDtypeStruct((B, 2, ni, H), jnp.float32),
        ),
        mesh=mesh,
        scratch_types=[
            pltpu.VMEM_SHARED((nu, H), jnp.float32),
            pltpu.VMEM((CH, H), jnp.float32),
            pltpu.VMEM((3, CH), jnp.int32),
            pltpu.VMEM((ZR, H), jnp.float32),
            pltpu.SemaphoreType.DMA,
        ],
        compiler_params=pltpu.CompilerParams(use_tc_tiling_on_sc=False,
                                             needs_layout_passes=False),
    )
    return f(p_item, p_user, pack)


def _post_body(a0_ref, a1_ref, embs_ref, emb_ref):
    a = jnp.concatenate([a0_ref[:, 0], a1_ref[:, 0]], axis=-1)
    embs_ref[...] = jax.nn.sigmoid(a)
    emb_ref[...] = jax.nn.sigmoid(jnp.mean(a, axis=0))


def _post(agg, block=2000):
    v = agg.shape[2]
    return pl.pallas_call(
        _post_body,
        grid=(v // block,),
        in_specs=[
            pl.BlockSpec((B, 1, block, H), lambda i: (0, 0, i, 0)),
            pl.BlockSpec((B, 1, block, H), lambda i: (0, 1, i, 0)),
        ],
        out_specs=[
            pl.BlockSpec((B, block, D), lambda i: (0, i, 0)),
            pl.BlockSpec((block, D), lambda i: (i, 0)),
        ],
        out_shape=[
            jax.ShapeDtypeStruct((B, v, D), jnp.float32),
            jax.ShapeDtypeStruct((v, D), jnp.float32),
        ],
    )(agg, agg)


def kernel(user_embedding, item_embedding, u_w, i_w, edge_user, edge_item,
           edge_val):
    p_item = _project(item_embedding, u_w)
    p_user = _project(user_embedding, i_w)
    agg_u, agg_i = _sc_aggregate(p_item, p_user, edge_user, edge_item,
                                 edge_val)
    user_embs, user_emb = _post(agg_u)
    item_embs, item_emb = _post(agg_i)
    return (user_emb, item_emb, user_embs, item_embs)


# speedup vs baseline: 1.2836x; 1.0304x over previous
import functools

import jax
import jax.numpy as jnp
from jax import lax
from jax.experimental import pallas as pl
from jax.experimental.pallas import tpu as pltpu
from jax.experimental.pallas import tpu_sc as plsc

B = 3
D = 64
H = 32
W = 128
CH = 400
NJ = 79
NTILES = 16
EPAD = NTILES * NJ * CH
SL = 3120
ZR = 80


def _proj_body(x_ref, w_ref, o_ref):
    res = jnp.dot(x_ref[...], w_ref[...], preferred_element_type=jnp.float32)
    o_ref[0] = res[:, :H]
    o_ref[1] = res[:, H:]


def _project(x, w, block=2000):
    v = x.shape[0]
    return pl.pallas_call(
        _proj_body,
        grid=(v // block,),
        in_specs=[
            pl.BlockSpec((block, D), lambda i: (i, 0)),
            pl.BlockSpec((D, D), lambda i: (0, 0)),
        ],
        out_specs=pl.BlockSpec((2, block, H), lambda i: (0, i, 0)),
        out_shape=jax.ShapeDtypeStruct((2, v, H), jnp.float32),
    )(x, w)


def _sc_body(nv, p_item, p_user, pack, agg_u, agg_i,
             acc, rows, pk, zeros, sem):
    c = lax.axis_index("c")
    s = lax.axis_index("s")
    rem = nv - NTILES * SL

    def zfill(r, zc):
        z = jnp.zeros((16,), jnp.float32)
        zeros[r, 0:16] = z
        zeros[r, 16:32] = z
        return zc
    lax.fori_loop(0, ZR, zfill, 0)

    def one_pass(b, table, splane, dplane, out):
        for k in range(SL // ZR):
            pltpu.sync_copy(zeros, acc.at[pl.ds(s * SL + k * ZR, ZR)])
        if rem:
            @pl.when(s == 0)
            def _():
                pltpu.sync_copy(zeros.at[pl.ds(0, rem)],
                                acc.at[pl.ds(NTILES * SL, rem)])
        plsc.subcore_barrier()

        pltpu.sync_copy(pack.at[b, s], pk.at[0])
        pltpu.async_copy(table.at[c].at[pk.at[0, splane]], rows.at[0], sem)

        def chunk_body(j, carry):
            q = j % 2

            @pl.when(j < NJ - 1)
            def _():
                pltpu.sync_copy(pack.at[b, s + NTILES * (j + 1)],
                                pk.at[1 - q])
                pltpu.async_copy(table.at[c].at[pk.at[1 - q, splane]],
                                 rows.at[1 - q], sem)

            pltpu.make_async_copy(table.at[c].at[pk.at[q, splane]],
                                  rows.at[q], sem).wait()

            def scale_body(g, sc_):
                vg = plsc.bitcast(pk[q, 2, pl.ds(g * 16, 16)], jnp.float32)
                base = g * 16
                for e in range(16):
                    bc = lax.gather(
                        vg, jnp.full((16, 1), e, jnp.int32),
                        _GATHER_DNUMS, (1,),
                        mode=lax.GatherScatterMode.PROMISE_IN_BOUNDS)
                    r = base + e
                    rows[q, r, 0:16] = rows[q, r, 0:16] * bc
                    rows[q, r, 16:32] = rows[q, r, 16:32] * bc
                return sc_
            lax.fori_loop(0, CH // 16, scale_body, 0)
            pltpu.sync_copy(rows.at[q], acc.at[pk.at[q, dplane]], add=True)
            return carry
        lax.fori_loop(0, NJ, chunk_body, 0)
        plsc.subcore_barrier()
        pltpu.sync_copy(acc.at[pl.ds(s * SL, SL)],
                        out.at[b, c, pl.ds(s * SL, SL)])
        if rem:
            @pl.when(s == 0)
            def _():
                pltpu.sync_copy(acc.at[pl.ds(NTILES * SL, rem)],
                                out.at[b, c, pl.ds(NTILES * SL, rem)])

    for b in range(B):
        one_pass(b, p_item, 1, 0, agg_u)
        one_pass(b, p_user, 0, 1, agg_i)


_GATHER_DNUMS = lax.GatherDimensionNumbers(
    offset_dims=(), collapsed_slice_dims=(0,), start_index_map=(0,))


def _sc_aggregate(p_item, p_user, e_user, e_item, e_val):
    nu = p_user.shape[1]
    ni = p_item.shape[1]
    ne0 = e_user.shape[1]
    pad = EPAD - ne0
    if pad:
        e_user = jnp.pad(e_user, ((0, 0), (0, pad)))
        e_item = jnp.pad(e_item, ((0, 0), (0, pad)))
        e_val = jnp.pad(e_val, ((0, 0), (0, pad)))
    nc = EPAD // CH
    pack = jnp.stack(
        [e_user.reshape(B, nc, CH),
         e_item.reshape(B, nc, CH),
         lax.bitcast_convert_type(e_val, jnp.int32).reshape(B, nc, CH)],
        axis=2)
    mesh = plsc.VectorSubcoreMesh(core_axis_name="c", subcore_axis_name="s")
    f = pl.kernel(
        functools.partial(_sc_body, nu),
        out_type=(
            jax.ShapeDtypeStruct((B, 2, nu, H), jnp.float32),
            jax.ShapeDtypeStruct((B, 2, ni, H), jnp.float32),
        ),
        mesh=mesh,
        scratch_types=[
            pltpu.VMEM_SHARED((nu, H), jnp.float32),
            pltpu.VMEM((2, CH, H), jnp.float32),
            pltpu.VMEM((2, 3, CH), jnp.int32),
            pltpu.VMEM((ZR, H), jnp.float32),
            pltpu.SemaphoreType.DMA,
        ],
        compiler_params=pltpu.CompilerParams(use_tc_tiling_on_sc=False,
                                             needs_layout_passes=False),
    )
    return f(p_item, p_user, pack)


def _post_body(a0_ref, a1_ref, embs_ref, emb_ref):
    a = jnp.concatenate([a0_ref[:, 0], a1_ref[:, 0]], axis=-1)
    embs_ref[...] = jax.nn.sigmoid(a)
    emb_ref[...] = jax.nn.sigmoid(jnp.mean(a, axis=0))


def _post(agg, block=2000):
    v = agg.shape[2]
    return pl.pallas_call(
        _post_body,
        grid=(v // block,),
        in_specs=[
            pl.BlockSpec((B, 1, block, H), lambda i: (0, 0, i, 0)),
            pl.BlockSpec((B, 1, block, H), lambda i: (0, 1, i, 0)),
        ],
        out_specs=[
            pl.BlockSpec((B, block, D), lambda i: (0, i, 0)),
            pl.BlockSpec((block, D), lambda i: (i, 0)),
        ],
        out_shape=[
            jax.ShapeDtypeStruct((B, v, D), jnp.float32),
            jax.ShapeDtypeStruct((v, D), jnp.float32),
        ],
    )(agg, agg)


def kernel(user_embedding, item_embedding, u_w, i_w, edge_user, edge_item,
           edge_val):
    p_item = _project(item_embedding, u_w)
    p_user = _project(user_embedding, i_w)
    agg_u, agg_i = _sc_aggregate(p_item, p_user, edge_user, edge_item,
                                 edge_val)
    user_embs, user_emb = _post(agg_u)
    item_embs, item_emb = _post(agg_i)
    return (user_emb, item_emb, user_embs, item_embs)
